# baseline (speedup 1.0000x reference)
import functools

import jax
import jax.numpy as jnp
from jax import lax
from jax.experimental import pallas as pl
from jax.experimental.pallas import tpu as pltpu
from jax.experimental.pallas import tpu_sc as plsc

_SIZE = 1024
_SCALE = 1024.0
_MAGIC = 8388608.0
_NC = 2
_NS = 16
_LANES = 16


def _make_sc_call(batch, h, w, rows_per_chunk, out_shape):
    hw = h * w
    n_total = batch * hw
    nw = _NC * _NS
    per_w = n_total // nw
    chunk = rows_per_chunk * w
    n_chunks = per_w // chunk
    mesh = plsc.VectorSubcoreMesh(
        core_axis_name="c", subcore_axis_name="s",
        num_cores=_NC, num_subcores=_NS)

    @functools.partial(
        pl.kernel,
        mesh=mesh,
        compiler_params=pltpu.CompilerParams(needs_layout_passes=False),
        out_type=jax.ShapeDtypeStruct(out_shape, jnp.float32),
        scratch_types=[
            pltpu.VMEM((_SIZE,), jnp.float32),
            pltpu.VMEM((_SIZE,), jnp.float32),
            pltpu.VMEM((_SIZE,), jnp.float32),
            (pltpu.VMEM((rows_per_chunk, w), jnp.float32),) * 2,
            (pltpu.VMEM((rows_per_chunk, w), jnp.float32),) * 2,
            (pltpu.VMEM((rows_per_chunk, w), jnp.float32),) * 2,
            (pltpu.VMEM((rows_per_chunk, w), jnp.float32),) * 2,
            pltpu.SemaphoreType.DMA,
            (pltpu.SemaphoreType.DMA,) * 2,
            (pltpu.SemaphoreType.DMA,) * 2,
        ],
    )
    def sc_colormap(x_nat, pal_hbm, out_nat, pal_r, pal_g, pal_b,
                    xring, o_r, o_g, o_b, pal_sem, in_sems, out_sems):
        x_hbm = x_nat.reshape(batch * h, w)
        out_hbm = out_nat.reshape(batch * 3 * h, w)
        cid = lax.axis_index("c")
        sid = lax.axis_index("s")
        wid = cid * _NS + sid
        pal_descs = [
            pltpu.async_copy(pal_hbm.at[pl.ds(c * _SIZE, _SIZE)], dst, pal_sem)
            for c, dst in enumerate((pal_r, pal_g, pal_b))]

        in_base = wid * per_w
        img = wid * per_w // hw
        rem_rows = (in_base - img * hw) // w
        in_row = pl.multiple_of(img * h + rem_rows, rows_per_chunk)
        out_row = pl.multiple_of(img * 3 * h + rem_rows, rows_per_chunk)

        xbufs = list(xring)
        obufs = [[o_r[p], o_g[p], o_b[p]] for p in (0, 1)]

        def start_in(j, p):
            row = pl.multiple_of(in_row + j * rows_per_chunk, rows_per_chunk)
            pltpu.async_copy(
                x_hbm.at[pl.ds(row, rows_per_chunk)], xbufs[p], in_sems[p])

        def start_out(j, p):
            row = pl.multiple_of(out_row + j * rows_per_chunk, rows_per_chunk)
            for c in range(3):
                pltpu.async_copy(
                    obufs[p][c],
                    out_hbm.at[pl.ds(row + c * h, rows_per_chunk)],
                    out_sems[p])

        def wait_in(p):
            pltpu.make_async_copy(
                x_hbm.at[pl.ds(0, rows_per_chunk)], xbufs[p],
                in_sems[p]).wait()

        def wait_out(p):
            for c in range(3):
                pltpu.make_async_copy(
                    obufs[p][c], out_hbm.at[pl.ds(0, rows_per_chunk)],
                    out_sems[p]).wait()

        def compute(p):
            xb = xbufs[p]
            ob = obufs[p]

            @plsc.parallel_loop(0, rows_per_chunk, step=1, unroll=2)
            def _(r):
                for c in range(w // _LANES):
                    sl = pl.ds(c * _LANES, _LANES)
                    t = xb[r, sl] * _SCALE
                    rr = (t + _MAGIC) - _MAGIC
                    rr = jnp.minimum(jnp.maximum(rr, 0.0), float(_SIZE - 1))
                    idx = rr.astype(jnp.int32)
                    ob[0][r, sl] = plsc.load_gather(pal_r, [idx])
                    ob[1][r, sl] = plsc.load_gather(pal_g, [idx])
                    ob[2][r, sl] = plsc.load_gather(pal_b, [idx])

        start_in(0, 0)
        start_in(1, 1)
        for d in pal_descs:
            d.wait()

        def pair_body(g, carry):
            for p in (0, 1):
                j = 2 * g + p
                wait_in(p)

                @pl.when(g > 0)
                def _():
                    wait_out(p)

                compute(p)
                start_out(j, p)

                @pl.when(j + 2 < n_chunks)
                def _():
                    start_in(j + 2, p)

            return carry

        lax.fori_loop(0, n_chunks // 2, pair_body, 0)
        wait_out(0)
        wait_out(1)

    return sc_colormap


def kernel(x, palette):
    b, h, w = x.shape
    call = _make_sc_call(b, h, w, 16, (b, 3, h, w))
    pal_t = palette.T.reshape(-1).astype(jnp.float32)
    return call(x, pal_t)


# speedup vs baseline: 1.7970x; 1.7970x over previous
import functools

import jax
import jax.numpy as jnp
from jax import lax
from jax.experimental import pallas as pl
from jax.experimental.pallas import tpu as pltpu
from jax.experimental.pallas import tpu_sc as plsc

_SIZE = 1024
_SCALE = 1024.0
_MAGIC = 8388608.0
_NC = 2
_NS = 16
_LANES = 16


def _make_sc_call(batch, h, w, rows_per_chunk, out_shape):
    hw = h * w
    n_total = batch * hw
    nw = _NC * _NS
    per_w = n_total // nw
    chunk = rows_per_chunk * w
    n_chunks = per_w // chunk
    mesh = plsc.VectorSubcoreMesh(
        core_axis_name="c", subcore_axis_name="s",
        num_cores=_NC, num_subcores=_NS)

    @functools.partial(
        pl.kernel,
        mesh=mesh,
        compiler_params=pltpu.CompilerParams(needs_layout_passes=False),
        out_type=jax.ShapeDtypeStruct(out_shape, jnp.float32),
        scratch_types=[
            pltpu.VMEM((_SIZE,), jnp.float32),
            pltpu.VMEM((_SIZE,), jnp.float32),
            pltpu.VMEM((_SIZE,), jnp.float32),
            (pltpu.VMEM((rows_per_chunk, w), jnp.float32),) * 2,
            (pltpu.VMEM((rows_per_chunk, w), jnp.float32),) * 2,
            (pltpu.VMEM((rows_per_chunk, w), jnp.float32),) * 2,
            (pltpu.VMEM((rows_per_chunk, w), jnp.float32),) * 2,
            pltpu.SemaphoreType.DMA,
            (pltpu.SemaphoreType.DMA,) * 2,
            (pltpu.SemaphoreType.DMA,) * 2,
        ],
    )
    def sc_colormap(x_nat, pal_hbm, out_nat, pal_r, pal_g, pal_b,
                    xring, o_r, o_g, o_b, pal_sem, in_sems, out_sems):
        x_hbm = x_nat.reshape(batch * h, w)
        out_hbm = out_nat.reshape(batch * 3 * h, w)
        cid = lax.axis_index("c")
        sid = lax.axis_index("s")
        wid = cid * _NS + sid
        pal_descs = [
            pltpu.async_copy(pal_hbm.at[pl.ds(c * _SIZE, _SIZE)], dst, pal_sem)
            for c, dst in enumerate((pal_r, pal_g, pal_b))]

        in_base = wid * per_w
        img = wid * per_w // hw
        rem_rows = (in_base - img * hw) // w
        in_row = pl.multiple_of(img * h + rem_rows, rows_per_chunk)
        out_row = pl.multiple_of(img * 3 * h + rem_rows, rows_per_chunk)

        xbufs = list(xring)
        obufs = [[o_r[p], o_g[p], o_b[p]] for p in (0, 1)]

        def start_in(j, p):
            row = pl.multiple_of(in_row + j * rows_per_chunk, rows_per_chunk)
            pltpu.async_copy(
                x_hbm.at[pl.ds(row, rows_per_chunk)], xbufs[p], in_sems[p])

        def start_out(j, p):
            row = pl.multiple_of(out_row + j * rows_per_chunk, rows_per_chunk)
            for c in range(3):
                pltpu.async_copy(
                    obufs[p][c],
                    out_hbm.at[pl.ds(row + c * h, rows_per_chunk)],
                    out_sems[p])

        def wait_in(p):
            pltpu.make_async_copy(
                x_hbm.at[pl.ds(0, rows_per_chunk)], xbufs[p],
                in_sems[p]).wait()

        def wait_out(p):
            for c in range(3):
                pltpu.make_async_copy(
                    obufs[p][c], out_hbm.at[pl.ds(0, rows_per_chunk)],
                    out_sems[p]).wait()

        def compute(p):
            xb = xbufs[p]
            ob = obufs[p]

            def row_body(r, carry):
                @plsc.parallel_loop(0, w, step=_LANES, unroll=8)
                def _(i):
                    sl = pl.ds(i, _LANES)
                    t = xb[r, sl] * _SCALE
                    rr = (t + _MAGIC) - _MAGIC
                    rr = jnp.minimum(jnp.maximum(rr, 0.0), float(_SIZE - 1))
                    idx = rr.astype(jnp.int32)
                    ob[0][r, sl] = plsc.load_gather(pal_r, [idx])
                    ob[1][r, sl] = plsc.load_gather(pal_g, [idx])
                    ob[2][r, sl] = plsc.load_gather(pal_b, [idx])
                return carry

            lax.fori_loop(0, rows_per_chunk, row_body, 0)

        start_in(0, 0)
        start_in(1, 1)
        for d in pal_descs:
            d.wait()

        def pair_body(g, carry):
            for p in (0, 1):
                j = 2 * g + p
                wait_in(p)

                @pl.when(g > 0)
                def _():
                    wait_out(p)

                compute(p)
                start_out(j, p)

                @pl.when(j + 2 < n_chunks)
                def _():
                    start_in(j + 2, p)

            return carry

        lax.fori_loop(0, n_chunks // 2, pair_body, 0)
        wait_out(0)
        wait_out(1)

    return sc_colormap


def kernel(x, palette):
    b, h, w = x.shape
    call = _make_sc_call(b, h, w, 16, (b, 3, h, w))
    pal_t = palette.T.reshape(-1).astype(jnp.float32)
    return call(x, pal_t)
